# baseline (speedup 1.0000x reference)
import functools

import jax
import jax.numpy as jnp
import numpy as np
from jax import lax
from jax.experimental import pallas as pl
from jax.experimental.pallas import tpu as pltpu
from jax.experimental.pallas import tpu_sc as plsc

_DIM = 256
_NH = 8
_M = 128
_TOPK = 8
_DPOS = 3
_N = 4096
_NB = _N // _M
_EH = _DIM // _NH
_NEG = np.float32(-1e5)
_SCALE = np.float32(1.0 / np.sqrt(_EH))
_TS = 512
_BQ = 256
_PROJ_CHUNK = 512
_AUG = _EH + _NB
_bf = jnp.bfloat16
_f32 = jnp.float32


def _dot_t(a, b):
    return jax.lax.dot_general(a, b, (((1,), (1,)), ((), ())),
                               preferred_element_type=_f32)


def _qprep_kernel(x_ref, pos_ref, wpet_ref, bpe_ref, wq_ref, bq_ref,
                  wk_ref, bk_ref, kbar_ref, kb_ref):
    i = pl.program_id(0)
    p = pos_ref[...]
    rel = (p - jnp.mean(p, axis=0, keepdims=True)).astype(_bf)
    pe = rel[:, 0:1].astype(_f32) * wpet_ref[0:1, :].astype(_f32)
    for d in range(1, _DPOS):
        pe = pe + rel[:, d:d + 1].astype(_f32) * wpet_ref[d:d + 1, :].astype(_f32)
    xp = (x_ref[...] + pe) + bpe_ref[...]
    xb = xp.astype(_bf)
    ind = (jax.lax.broadcasted_iota(jnp.int32, (_M, _NB), 1) == i).astype(_bf)
    for h in range(_NH):
        q = _dot_t(xb, wq_ref[h]) + bq_ref[h]
        kb_ref[h, :, :] = jnp.concatenate([q.astype(_bf), ind], axis=1)
    k = _dot_t(xb, wk_ref[...]) + bk_ref[...]
    kbar_ref[0, :, :] = jnp.mean(k, axis=0, keepdims=True)


def _sim_kernel(c_ref, kb_ref, sim_ref):
    centers = c_ref[0].astype(_bf)
    q = kb_ref[0][:, :_EH]
    sim_ref[0, :, :] = _dot_t(centers, q)


_SC_CHUNK = _N // 4
_SC_L = 16


def _sc_select_body(sim_hbm, bias_hbm, sim_v, bias_v):
    wid = lax.axis_index("s") * 2 + lax.axis_index("c")
    pltpu.sync_copy(sim_hbm.at[wid], sim_v)

    def group(g, carry):
        sl = pl.ds(g * _SC_L, _SC_L)
        vs = [sim_v[b, sl] for b in range(_NB)]
        t = [jnp.full((_SC_L,), -np.inf, _f32) for _ in range(_TOPK)]
        for b in range(_NB):
            cur = vs[b]
            for j in range(_TOPK):
                hi = jnp.maximum(t[j], cur)
                cur = jnp.minimum(t[j], cur)
                t[j] = hi
        t8 = t[_TOPK - 1]
        one, zero = np.float32(1.0), np.float32(0.0)
        ngt = jnp.zeros((_SC_L,), _f32)
        for b in range(_NB):
            ngt = ngt + jnp.where(vs[b] > t8, one, zero)
        rem = np.float32(_TOPK) - ngt
        for b in range(_NB):
            takeeq = jnp.logical_and(vs[b] == t8, rem > 0)
            take = jnp.logical_or(vs[b] > t8, takeeq)
            rem = rem - jnp.where(takeeq, one, zero)
            bias_v[b, sl] = jnp.where(take, zero, _NEG)
        return carry

    lax.fori_loop(0, _SC_CHUNK // _SC_L, group, 0)
    pltpu.sync_copy(bias_v, bias_hbm.at[wid])


_sc_select = functools.partial(
    pl.kernel,
    mesh=plsc.VectorSubcoreMesh(core_axis_name="c", subcore_axis_name="s"),
    out_type=jax.ShapeDtypeStruct((32, _NB, _SC_CHUNK), _f32),
    scratch_types=[
        pltpu.VMEM((_NB, _SC_CHUNK), _f32),
        pltpu.VMEM((_NB, _SC_CHUNK), _f32),
    ],
)(_sc_select_body)


def _attn_kernel(bias_ref, kb_ref, o_ref):
    ib = pl.program_id(1)
    kb = kb_ref[0]
    qrows = kb_ref[0, pl.ds(ib * _BQ, _BQ), 0:_EH]
    qs = (qrows.astype(_f32) * _SCALE).astype(_bf)
    qa = jnp.concatenate([qs, bias_ref[0]], axis=1)
    s = _dot_t(qa, kb)
    p = jnp.exp(s).astype(_bf)
    o = jax.lax.dot_general(
        p, kb, (((1,), (0,)), ((), ())), preferred_element_type=_f32)
    l = jnp.sum(o[:, _EH:], axis=1, keepdims=True)
    o_ref[0, :, :] = o[:, :_EH] / l


def _proj_kernel(a_ref, wp_ref, bp_ref, o_ref):
    acc = jnp.zeros((_PROJ_CHUNK, _DIM), _f32) + bp_ref[...]
    for h in range(_NH):
        acc = acc + jax.lax.dot_general(
            a_ref[h].astype(_bf), wp_ref[h], (((1,), (0,)), ((), ())),
            preferred_element_type=_f32)
    o_ref[...] = acc


def kernel(x, pos, W_qkv, b_qkv, W_proj, b_proj, W_pe, b_pe):
    Wq = W_qkv[0::3].reshape(_NH, _EH, _DIM).astype(_bf)
    bq = b_qkv[0::3].reshape(_NH, 1, _EH)
    Wk = W_qkv[1::3].astype(_bf)
    bk = b_qkv[1::3].reshape(1, _DIM)
    WpeT = W_pe.T.astype(_bf)
    bpe = b_pe.reshape(1, _DIM)
    Wp = W_proj.T.reshape(_NH, _EH, _DIM).astype(_bf)
    bp = b_proj.reshape(1, _DIM)

    kbar, kb = pl.pallas_call(
        _qprep_kernel,
        grid=(_NB,),
        in_specs=[
            pl.BlockSpec((_M, _DIM), lambda i: (i, 0)),
            pl.BlockSpec((_M, _DPOS), lambda i: (i, 0)),
            pl.BlockSpec((_DPOS, _DIM), lambda i: (0, 0)),
            pl.BlockSpec((1, _DIM), lambda i: (0, 0)),
            pl.BlockSpec((_NH, _EH, _DIM), lambda i: (0, 0, 0)),
            pl.BlockSpec((_NH, 1, _EH), lambda i: (0, 0, 0)),
            pl.BlockSpec((_DIM, _DIM), lambda i: (0, 0)),
            pl.BlockSpec((1, _DIM), lambda i: (0, 0)),
        ],
        out_specs=[
            pl.BlockSpec((1, 1, _DIM), lambda i: (i, 0, 0)),
            pl.BlockSpec((_NH, _M, _AUG), lambda i: (0, i, 0)),
        ],
        out_shape=[
            jax.ShapeDtypeStruct((_NB, 1, _DIM), _f32),
            jax.ShapeDtypeStruct((_NH, _N, _AUG), _bf),
        ],
    )(x, pos, WpeT, bpe, Wq, bq, Wk, bk)

    centers = jnp.transpose(kbar.reshape(_NB, _NH, _EH), (1, 0, 2))

    sim = pl.pallas_call(
        _sim_kernel,
        grid=(_NH, _N // _TS),
        in_specs=[
            pl.BlockSpec((1, _NB, _EH), lambda h, c: (h, 0, 0)),
            pl.BlockSpec((1, _TS, _AUG), lambda h, c: (h, c, 0)),
        ],
        out_specs=pl.BlockSpec((1, _NB, _TS), lambda h, c: (h, 0, c)),
        out_shape=jax.ShapeDtypeStruct((_NH, _NB, _N), _f32),
    )(centers, kb)

    sim4 = sim.reshape(_NH, _NB, 4, _SC_CHUNK).transpose(0, 2, 1, 3)
    sim4 = sim4.reshape(32, _NB, _SC_CHUNK)
    bias4 = _sc_select(sim4)
    bias_bm = bias4.reshape(_NH, 4, _NB, _SC_CHUNK).transpose(0, 2, 1, 3)
    bias_bm = bias_bm.reshape(_NH, _NB, _N)

    bias_tok = jnp.transpose(bias_bm, (0, 2, 1)).astype(_bf)

    attn = pl.pallas_call(
        _attn_kernel,
        grid=(_NH, _N // _BQ),
        in_specs=[
            pl.BlockSpec((1, _BQ, _NB), lambda h, i: (h, i, 0)),
            pl.BlockSpec((1, _N, _AUG), lambda h, i: (h, 0, 0)),
        ],
        out_specs=pl.BlockSpec((1, _BQ, _EH), lambda h, i: (h, i, 0)),
        out_shape=jax.ShapeDtypeStruct((_NH, _N, _EH), _f32),
    )(bias_tok, kb)

    out = pl.pallas_call(
        _proj_kernel,
        grid=(_N // _PROJ_CHUNK,),
        in_specs=[
            pl.BlockSpec((_NH, _PROJ_CHUNK, _EH), lambda r: (0, r, 0)),
            pl.BlockSpec((_NH, _EH, _DIM), lambda r: (0, 0, 0)),
            pl.BlockSpec((1, _DIM), lambda r: (0, 0)),
        ],
        out_specs=pl.BlockSpec((_PROJ_CHUNK, _DIM), lambda r: (r, 0)),
        out_shape=jax.ShapeDtypeStruct((_N, _DIM), _f32),
    )(attn, Wp, bp)

    return out


# speedup vs baseline: 1.0162x; 1.0162x over previous
import functools

import jax
import jax.numpy as jnp
import numpy as np
from jax import lax
from jax.experimental import pallas as pl
from jax.experimental.pallas import tpu as pltpu
from jax.experimental.pallas import tpu_sc as plsc

_DIM = 256
_NH = 8
_M = 128
_TOPK = 8
_DPOS = 3
_N = 4096
_NB = _N // _M
_EH = _DIM // _NH
_NEG = np.float32(-1e5)
_SCALE = np.float32(1.0 / np.sqrt(_EH))
_TS = 512
_BQ = 256
_PROJ_CHUNK = 512
_AUG = _EH + _NB
_bf = jnp.bfloat16
_f32 = jnp.float32


def _dot_t(a, b):
    return jax.lax.dot_general(a, b, (((1,), (1,)), ((), ())),
                               preferred_element_type=_f32)


def _qprep_kernel(x_ref, pos_ref, wpet_ref, bpe_ref, wq_ref, bq_ref,
                  wk_ref, bk_ref, kbar_ref, kb_ref):
    i = pl.program_id(0)
    p = pos_ref[...]
    rel = (p - jnp.mean(p, axis=0, keepdims=True)).astype(_bf)
    pe = rel[:, 0:1].astype(_f32) * wpet_ref[0:1, :].astype(_f32)
    for d in range(1, _DPOS):
        pe = pe + rel[:, d:d + 1].astype(_f32) * wpet_ref[d:d + 1, :].astype(_f32)
    xp = (x_ref[...] + pe) + bpe_ref[...]
    xb = xp.astype(_bf)
    ind = (jax.lax.broadcasted_iota(jnp.int32, (_M, _NB), 1) == i).astype(_bf)
    for h in range(_NH):
        q = _dot_t(xb, wq_ref[h]) + bq_ref[h]
        kb_ref[h, :, :] = jnp.concatenate([q.astype(_bf), ind], axis=1)
    k = _dot_t(xb, wk_ref[...]) + bk_ref[...]
    kbar_ref[0, :, :] = jnp.mean(k, axis=0, keepdims=True)


def _sim_kernel(c_ref, kb_ref, sim_ref):
    centers = c_ref[0].astype(_bf)
    q = kb_ref[0][:, :_EH]
    sim_ref[0, :, :] = _dot_t(centers, q)


_SC_CHUNK = _N // 4
_SC_L = 16


def _sc_select_body(sim_hbm, bias_hbm, sim_v, bias_v):
    wid = lax.axis_index("s") * 2 + lax.axis_index("c")
    pltpu.sync_copy(sim_hbm.at[wid], sim_v)

    def group(g, carry):
        sl = pl.ds(g * _SC_L, _SC_L)
        vs = [sim_v[b, sl] for b in range(_NB)]
        t = [jnp.full((_SC_L,), -np.inf, _f32) for _ in range(_TOPK)]
        for b in range(_NB):
            cur = vs[b]
            for j in range(_TOPK):
                hi = jnp.maximum(t[j], cur)
                cur = jnp.minimum(t[j], cur)
                t[j] = hi
        t8 = t[_TOPK - 1]
        one, zero = np.float32(1.0), np.float32(0.0)
        ngt = jnp.zeros((_SC_L,), _f32)
        for b in range(_NB):
            ngt = ngt + jnp.where(vs[b] > t8, one, zero)
        rem = np.float32(_TOPK) - ngt
        for b in range(_NB):
            takeeq = jnp.logical_and(vs[b] == t8, rem > 0)
            take = jnp.logical_or(vs[b] > t8, takeeq)
            rem = rem - jnp.where(takeeq, one, zero)
            bias_v[b, sl] = jnp.where(take, zero, _NEG)
        return carry

    lax.fori_loop(0, _SC_CHUNK // _SC_L, group, 0)
    pltpu.sync_copy(bias_v, bias_hbm.at[wid])


_sc_select = functools.partial(
    pl.kernel,
    mesh=plsc.VectorSubcoreMesh(core_axis_name="c", subcore_axis_name="s"),
    out_type=jax.ShapeDtypeStruct((32, _NB, _SC_CHUNK), _f32),
    scratch_types=[
        pltpu.VMEM((_NB, _SC_CHUNK), _f32),
        pltpu.VMEM((_NB, _SC_CHUNK), _f32),
    ],
)(_sc_select_body)


def _attn_kernel(bias_ref, kb_ref, o_ref):
    ib = pl.program_id(1)
    kb = kb_ref[0]
    qrows = kb_ref[0, pl.ds(ib * _BQ, _BQ), 0:_EH]
    qs = (qrows.astype(_f32) * _SCALE).astype(_bf)
    qa = jnp.concatenate([qs, bias_ref[0]], axis=1)
    s = _dot_t(qa, kb)
    p = jnp.exp(s).astype(_bf)
    o = jax.lax.dot_general(
        p, kb, (((1,), (0,)), ((), ())), preferred_element_type=_f32)
    l = jnp.sum(o[:, _EH:], axis=1, keepdims=True)
    o_ref[0, :, :] = o[:, :_EH] / l


def _proj_kernel(a_ref, wp_ref, bp_ref, o_ref):
    acc = jnp.zeros((_PROJ_CHUNK, _DIM), _f32) + bp_ref[...]
    for h in range(_NH):
        acc = acc + jax.lax.dot_general(
            a_ref[h].astype(_bf), wp_ref[h], (((1,), (0,)), ((), ())),
            preferred_element_type=_f32)
    o_ref[...] = acc


def kernel(x, pos, W_qkv, b_qkv, W_proj, b_proj, W_pe, b_pe):
    Wq = W_qkv[0::3].reshape(_NH, _EH, _DIM).astype(_bf)
    bq = b_qkv[0::3].reshape(_NH, 1, _EH)
    Wk = W_qkv[1::3].astype(_bf)
    bk = b_qkv[1::3].reshape(1, _DIM)
    WpeT = W_pe.T.astype(_bf)
    bpe = b_pe.reshape(1, _DIM)
    Wp = W_proj.T.reshape(_NH, _EH, _DIM).astype(_bf)
    bp = b_proj.reshape(1, _DIM)

    kbar, kb = pl.pallas_call(
        _qprep_kernel,
        grid=(_NB,),
        in_specs=[
            pl.BlockSpec((_M, _DIM), lambda i: (i, 0)),
            pl.BlockSpec((_M, _DPOS), lambda i: (i, 0)),
            pl.BlockSpec((_DPOS, _DIM), lambda i: (0, 0)),
            pl.BlockSpec((1, _DIM), lambda i: (0, 0)),
            pl.BlockSpec((_NH, _EH, _DIM), lambda i: (0, 0, 0)),
            pl.BlockSpec((_NH, 1, _EH), lambda i: (0, 0, 0)),
            pl.BlockSpec((_DIM, _DIM), lambda i: (0, 0)),
            pl.BlockSpec((1, _DIM), lambda i: (0, 0)),
        ],
        out_specs=[
            pl.BlockSpec((1, 1, _DIM), lambda i: (i, 0, 0)),
            pl.BlockSpec((_NH, _M, _AUG), lambda i: (0, i, 0)),
        ],
        out_shape=[
            jax.ShapeDtypeStruct((_NB, 1, _DIM), _f32),
            jax.ShapeDtypeStruct((_NH, _N, _AUG), _bf),
        ],
    )(x, pos, WpeT, bpe, Wq, bq, Wk, bk)

    centers = jnp.transpose(kbar.reshape(_NB, _NH, _EH), (1, 0, 2))

    _TPW = _SC_CHUNK // _TS
    sim4 = pl.pallas_call(
        _sim_kernel,
        grid=(_NH, _N // _TS),
        in_specs=[
            pl.BlockSpec((1, _NB, _EH), lambda h, c: (h, 0, 0)),
            pl.BlockSpec((1, _TS, _AUG), lambda h, c: (h, c, 0)),
        ],
        out_specs=pl.BlockSpec((1, _NB, _TS),
                               lambda h, c: (h * 4 + c // _TPW, 0, c % _TPW)),
        out_shape=jax.ShapeDtypeStruct((32, _NB, _SC_CHUNK), _f32),
    )(centers, kb)

    bias4 = _sc_select(sim4)
    bias_tok = (bias4.reshape(_NH, 4, _NB, _SC_CHUNK)
                .transpose(0, 1, 3, 2)
                .reshape(_NH, _N, _NB).astype(_bf))

    attn = pl.pallas_call(
        _attn_kernel,
        grid=(_NH, _N // _BQ),
        in_specs=[
            pl.BlockSpec((1, _BQ, _NB), lambda h, i: (h, i, 0)),
            pl.BlockSpec((1, _N, _AUG), lambda h, i: (h, 0, 0)),
        ],
        out_specs=pl.BlockSpec((1, _BQ, _EH), lambda h, i: (h, i, 0)),
        out_shape=jax.ShapeDtypeStruct((_NH, _N, _EH), _f32),
    )(bias_tok, kb)

    out = pl.pallas_call(
        _proj_kernel,
        grid=(_N // _PROJ_CHUNK,),
        in_specs=[
            pl.BlockSpec((_NH, _PROJ_CHUNK, _EH), lambda r: (0, r, 0)),
            pl.BlockSpec((_NH, _EH, _DIM), lambda r: (0, 0, 0)),
            pl.BlockSpec((1, _DIM), lambda r: (0, 0)),
        ],
        out_specs=pl.BlockSpec((_PROJ_CHUNK, _DIM), lambda r: (r, 0)),
        out_shape=jax.ShapeDtypeStruct((_N, _DIM), _f32),
    )(attn, Wp, bp)

    return out
